# baseline (speedup 1.0000x reference)
import jax
import jax.numpy as jnp
from jax.experimental import pallas as pl
from jax.experimental.pallas import tpu as pltpu


_BM = 256


def _mm_body(x_ref, w_ref, o_ref):
    wb = w_ref[...].astype(jnp.bfloat16)
    m = x_ref.shape[0]
    for i in range(m // _BM):
        xs = pl.ds(i * _BM, _BM)
        o_ref[xs, :] = jax.lax.dot_general(
            x_ref[xs, :],
            wb,
            dimension_numbers=(((1,), (1,)), ((), ())),
            preferred_element_type=jnp.float32,
        )


def kernel(x, weight):
    b, s, d_model = x.shape
    d_ff = weight.shape[0]
    m = b * s

    x2 = x.reshape(m, d_model)

    bn = 256
    out = pl.pallas_call(
        _mm_body,
        grid=(d_ff // bn,),
        in_specs=[
            pl.BlockSpec((m, d_model), lambda j: (0, 0)),
            pl.BlockSpec((bn, d_model), lambda j: (j, 0)),
        ],
        out_specs=pl.BlockSpec((m, bn), lambda j: (0, j)),
        out_shape=jax.ShapeDtypeStruct((m, d_ff), jnp.float32),
        compiler_params=pltpu.CompilerParams(
            dimension_semantics=("parallel",),
            vmem_limit_bytes=100 * 1024 * 1024,
        ),
    )(x2, weight)
    return out.reshape(b, s, d_ff)


# speedup vs baseline: 1.0349x; 1.0349x over previous
import jax
import jax.numpy as jnp
from jax.experimental import pallas as pl
from jax.experimental.pallas import tpu as pltpu


_BM = 512


def _mm_body(x_ref, w_ref, o_ref):
    wb = w_ref[...].astype(jnp.bfloat16)
    m = x_ref.shape[0]
    for i in range(m // _BM):
        xs = pl.ds(i * _BM, _BM)
        o_ref[xs, :] = jax.lax.dot_general(
            x_ref[xs, :],
            wb,
            dimension_numbers=(((1,), (1,)), ((), ())),
            preferred_element_type=jnp.float32,
        )


def kernel(x, weight):
    b, s, d_model = x.shape
    d_ff = weight.shape[0]
    m = b * s

    x2 = x.reshape(m, d_model)

    bn = 512
    out = pl.pallas_call(
        _mm_body,
        grid=(d_ff // bn,),
        in_specs=[
            pl.BlockSpec((m, d_model), lambda j: (0, 0)),
            pl.BlockSpec((bn, d_model), lambda j: (j, 0)),
        ],
        out_specs=pl.BlockSpec((m, bn), lambda j: (0, j)),
        out_shape=jax.ShapeDtypeStruct((m, d_ff), jnp.float32),
        compiler_params=pltpu.CompilerParams(
            dimension_semantics=("parallel",),
            vmem_limit_bytes=100 * 1024 * 1024,
        ),
    )(x2, weight)
    return out.reshape(b, s, d_ff)


# speedup vs baseline: 1.0369x; 1.0019x over previous
import jax
import jax.numpy as jnp
from jax.experimental import pallas as pl
from jax.experimental.pallas import tpu as pltpu


_BM = 1024


def _mm_body(x_ref, w_ref, o_ref):
    wb = w_ref[...].astype(jnp.bfloat16)
    m = x_ref.shape[0]
    for i in range(m // _BM):
        xs = pl.ds(i * _BM, _BM)
        o_ref[xs, :] = jax.lax.dot_general(
            x_ref[xs, :],
            wb,
            dimension_numbers=(((1,), (1,)), ((), ())),
            preferred_element_type=jnp.float32,
        )


def kernel(x, weight):
    b, s, d_model = x.shape
    d_ff = weight.shape[0]
    m = b * s

    x2 = x.reshape(m, d_model)

    bn = 512
    out = pl.pallas_call(
        _mm_body,
        grid=(d_ff // bn,),
        in_specs=[
            pl.BlockSpec((m, d_model), lambda j: (0, 0)),
            pl.BlockSpec((bn, d_model), lambda j: (j, 0)),
        ],
        out_specs=pl.BlockSpec((m, bn), lambda j: (0, j)),
        out_shape=jax.ShapeDtypeStruct((m, d_ff), jnp.float32),
        compiler_params=pltpu.CompilerParams(
            dimension_semantics=("parallel",),
            vmem_limit_bytes=100 * 1024 * 1024,
        ),
    )(x2, weight)
    return out.reshape(b, s, d_ff)
